# baseline (speedup 1.0000x reference)
import functools

import jax
import jax.numpy as jnp
from jax import lax
from jax.experimental import pallas as pl
from jax.experimental.pallas import tpu as pltpu
from jax.experimental.pallas import tpu_sc as plsc

_B, _C, _N, _K = 8, 96, 1024, 9
_CP = 128
_KP = 16
_M = 256
_R = 256


def _prep_body(xf_ref, xc_ref, a_ref, bm_ref, b1_ref, idx_ref, p_ref, q_ref):
    b = pl.program_id(0)
    xf = xf_ref[0]
    xc = xc_ref[0]
    sq = jnp.sum(xf * xf, axis=0, keepdims=True)
    g = lax.dot_general(xc, xf, (((0,), (0,)), ((), ())),
                        preferred_element_type=jnp.float32)
    score = sq - 2.0 * g
    iota_f = lax.broadcasted_iota(jnp.int32, (_M, _N), 1).astype(jnp.float32)
    cols = []
    m = jnp.min(score, axis=1, keepdims=True)
    for j in range(_K):
        cmp = score == m
        cols.append(jnp.min(jnp.where(cmp, iota_f, 65536.0), axis=1,
                            keepdims=True).astype(jnp.int32))
        if j + 1 < _K:
            score = jnp.where(cmp, jnp.inf, score)
            m = jnp.min(score, axis=1, keepdims=True)
    idx_ref[0] = jnp.concatenate(cols, axis=1) + b * _N
    wc = a_ref[...] - bm_ref[...]
    p_ref[0] = lax.dot_general(xc, wc, (((0,), (1,)), ((), ())),
                               preferred_element_type=jnp.float32) + b1_ref[...]
    qv = lax.dot_general(xc, bm_ref[...], (((0,), (1,)), ((), ())),
                         preferred_element_type=jnp.float32)
    q_ref[0] = jnp.concatenate(
        [qv, jnp.zeros((_M, _CP - _C), jnp.float32)], axis=1)


def _prep_call(xr, w1a, w1b, b1row, interpret=False):
    return pl.pallas_call(
        _prep_body,
        grid=(_B, _N // _M),
        in_specs=[
            pl.BlockSpec((1, _C, _N), lambda b, t: (b, 0, 0)),
            pl.BlockSpec((1, _C, _M), lambda b, t: (b, 0, t)),
            pl.BlockSpec((_C, _C), lambda b, t: (0, 0)),
            pl.BlockSpec((_C, _C), lambda b, t: (0, 0)),
            pl.BlockSpec((1, _C), lambda b, t: (0, 0)),
        ],
        out_specs=[
            pl.BlockSpec((1, _M, _K), lambda b, t: (b, t, 0)),
            pl.BlockSpec((1, _M, _C), lambda b, t: (b, t, 0)),
            pl.BlockSpec((1, _M, _CP), lambda b, t: (b, t, 0)),
        ],
        out_shape=[
            jax.ShapeDtypeStruct((_B, _N, _K), jnp.int32),
            jax.ShapeDtypeStruct((_B, _N, _C), jnp.float32),
            jax.ShapeDtypeStruct((_B, _N, _CP), jnp.float32),
        ],
        interpret=interpret,
    )(xr, xr, w1a, w1b, b1row)


_TOT = _B * _N * _K
_NW = 32
_PW = _TOT // _NW
_CH = 128
_NCH = _PW // _CH


def _gather_body(q_hbm, idx_hbm, out_hbm, idx_v, rows_v, sem):
    wid = lax.axis_index("s") * 2 + lax.axis_index("c")

    def step(t, carry):
        base = wid * _PW + t * _CH
        pltpu.sync_copy(idx_hbm.at[pl.ds(base, _CH)], idx_v)
        pltpu.async_copy(q_hbm.at[idx_v], rows_v, sem).wait()
        pltpu.sync_copy(rows_v, out_hbm.at[pl.ds(base, _CH)])
        return carry

    lax.fori_loop(0, _NCH, step, 0)


@functools.cache
def _gather_call():
    return pl.kernel(
        _gather_body,
        out_type=jax.ShapeDtypeStruct((_TOT, _CP), jnp.float32),
        mesh=plsc.VectorSubcoreMesh(core_axis_name="c", subcore_axis_name="s"),
        scratch_types=[
            pltpu.VMEM((_CH,), jnp.int32),
            pltpu.VMEM((_CH, _CP), jnp.float32),
            pltpu.SemaphoreType.DMA,
        ],
    )


def _mlp_body(qg_ref, p_ref, w2_ref, b2_ref, msg_ref):
    p = p_ref[0]
    w2 = w2_ref[...]
    acc = None
    for j in range(_K):
        v = p + qg_ref[0, :, j, :_C]
        v = jnp.where(v >= 0, v, 0.2 * v)
        hj = lax.dot_general(w2, v, (((1,), (1,)), ((), ())),
                             preferred_element_type=jnp.float32)
        acc = hj if acc is None else jnp.maximum(acc, hj)
    msg_ref[0] = acc + b2_ref[...]


def _mlp_call(qg, p, w2, b2col, interpret=False):
    return pl.pallas_call(
        _mlp_body,
        grid=(_B, _N // _R),
        in_specs=[
            pl.BlockSpec((1, _R, _K, _CP), lambda b, t: (b, t, 0, 0)),
            pl.BlockSpec((1, _R, _C), lambda b, t: (b, t, 0)),
            pl.BlockSpec((_C, _C), lambda b, t: (0, 0)),
            pl.BlockSpec((_C, 1), lambda b, t: (0, 0)),
        ],
        out_specs=pl.BlockSpec((1, _C, _R), lambda b, t: (b, 0, t)),
        out_shape=jax.ShapeDtypeStruct((_B, _C, _N), jnp.float32),
        interpret=interpret,
    )(qg, p, w2, b2col)


def _bn_body(msg_ref, x_ref, gamma_ref, beta_ref, out_ref):
    s = None
    for b in range(_B):
        sb = jnp.sum(msg_ref[b], axis=1, keepdims=True)
        s = sb if s is None else s + sb
    mean = s * (1.0 / (_B * _N))
    v = None
    for b in range(_B):
        c = msg_ref[b] - mean
        vb = jnp.sum(c * c, axis=1, keepdims=True)
        v = vb if v is None else v + vb
    var = v * (1.0 / (_B * _N))
    scale = lax.rsqrt(var + 1e-5) * gamma_ref[...]
    shift = beta_ref[...] - mean * scale
    for b in range(_B):
        out_ref[b] = jnp.maximum(msg_ref[b] * scale + shift + x_ref[b], 0.0)


def _bn_call(msg, xr, gammacol, betacol, interpret=False):
    return pl.pallas_call(
        _bn_body,
        out_shape=jax.ShapeDtypeStruct((_B, _C, _N), jnp.float32),
        interpret=interpret,
    )(msg, xr, gammacol, betacol)


def kernel(x, W1, b1, W2, b2, gamma, beta):
    B, C, H, W = x.shape
    xr = x.reshape(B, C, H * W)
    w1a = W1[:, :C]
    w1b = W1[:, C:]
    idx, p, q = _prep_call(xr, w1a, w1b, b1.reshape(1, C))
    qg = _gather_call()(q.reshape(B * H * W, _CP), idx.reshape(-1))
    msg = _mlp_call(qg.reshape(B, H * W, _K, _CP), p, W2, b2.reshape(C, 1))
    out = _bn_call(msg, xr, gamma.reshape(C, 1), beta.reshape(C, 1))
    return out.reshape(B, C, H, W)


# speedup vs baseline: 1.2877x; 1.2877x over previous
import functools

import jax
import jax.numpy as jnp
from jax import lax
from jax.experimental import pallas as pl
from jax.experimental.pallas import tpu as pltpu
from jax.experimental.pallas import tpu_sc as plsc

_B, _C, _N, _K = 8, 96, 1024, 9
_CP = 128
_KP = 16
_M = 256
_R = 256


def _prep_body(xf_ref, xc_ref, a_ref, bm_ref, b1_ref, idx_ref, p_ref, q_ref):
    b = pl.program_id(0)
    xf = xf_ref[0]
    xc = xc_ref[0]
    sq = jnp.sum(xf * xf, axis=0, keepdims=True)
    g = lax.dot_general(xc, xf, (((0,), (0,)), ((), ())),
                        preferred_element_type=jnp.float32)
    score = sq - 2.0 * g
    iota_f = lax.broadcasted_iota(jnp.int32, (_M, _N), 1).astype(jnp.float32)
    cols = []
    m = jnp.min(score, axis=1, keepdims=True)
    for j in range(_K):
        cmp = score == m
        cols.append(jnp.min(jnp.where(cmp, iota_f, 65536.0), axis=1,
                            keepdims=True).astype(jnp.int32))
        if j + 1 < _K:
            score = jnp.where(cmp, jnp.inf, score)
            m = jnp.min(score, axis=1, keepdims=True)
    idx_ref[0] = jnp.concatenate(cols, axis=1) + b * _N
    wc = a_ref[...] - bm_ref[...]
    p_ref[0] = lax.dot_general(xc, wc, (((0,), (1,)), ((), ())),
                               preferred_element_type=jnp.float32) + b1_ref[...]
    qv = lax.dot_general(xc, bm_ref[...], (((0,), (1,)), ((), ())),
                         preferred_element_type=jnp.float32)
    q_ref[0] = jnp.concatenate(
        [qv, jnp.zeros((_M, _CP - _C), jnp.float32)], axis=1)


def _prep_call(xr, w1a, w1b, b1row, interpret=False):
    return pl.pallas_call(
        _prep_body,
        grid=(_B, _N // _M),
        in_specs=[
            pl.BlockSpec((1, _C, _N), lambda b, t: (b, 0, 0)),
            pl.BlockSpec((1, _C, _M), lambda b, t: (b, 0, t)),
            pl.BlockSpec((_C, _C), lambda b, t: (0, 0)),
            pl.BlockSpec((_C, _C), lambda b, t: (0, 0)),
            pl.BlockSpec((1, _C), lambda b, t: (0, 0)),
        ],
        out_specs=[
            pl.BlockSpec((1, _M, _K), lambda b, t: (b, t, 0)),
            pl.BlockSpec((1, _M, _C), lambda b, t: (b, t, 0)),
            pl.BlockSpec((1, _M, _CP), lambda b, t: (b, t, 0)),
        ],
        out_shape=[
            jax.ShapeDtypeStruct((_B, _N, _K), jnp.int32),
            jax.ShapeDtypeStruct((_B, _N, _C), jnp.float32),
            jax.ShapeDtypeStruct((_B, _N, _CP), jnp.float32),
        ],
        interpret=interpret,
    )(xr, xr, w1a, w1b, b1row)


_TOT = _B * _N * _K
_NW = 32
_PW = _TOT // _NW
_CH = 128
_NCH = _PW // _CH


def _gather_body(q_hbm, idx_hbm, pos_hbm, out_hbm, idx_v, pos_v, rows_v, sem):
    wid = lax.axis_index("s") * 2 + lax.axis_index("c")

    def step(t, carry):
        base = wid * _PW + t * _CH
        pltpu.sync_copy(idx_hbm.at[pl.ds(base, _CH)], idx_v)
        pltpu.sync_copy(pos_hbm.at[pl.ds(base, _CH)], pos_v)
        pltpu.async_copy(q_hbm.at[idx_v], rows_v, sem).wait()
        pltpu.async_copy(rows_v, out_hbm.at[pos_v], sem).wait()
        return carry

    lax.fori_loop(0, _NCH, step, 0)


@functools.cache
def _gather_call():
    return pl.kernel(
        _gather_body,
        out_type=jax.ShapeDtypeStruct((_TOT, _CP), jnp.float32),
        mesh=plsc.VectorSubcoreMesh(core_axis_name="c", subcore_axis_name="s"),
        scratch_types=[
            pltpu.VMEM((_CH,), jnp.int32),
            pltpu.VMEM((_CH,), jnp.int32),
            pltpu.VMEM((_CH, _CP), jnp.float32),
            pltpu.SemaphoreType.DMA,
        ],
    )


def _mlp_body(qg_ref, p_ref, w2_ref, b2_ref, msg_ref):
    p = p_ref[0]
    w2 = w2_ref[...]
    acc = None
    for j in range(_K):
        v = p + qg_ref[0, j, :, :_C]
        v = jnp.where(v >= 0, v, 0.2 * v)
        hj = lax.dot_general(w2, v, (((1,), (1,)), ((), ())),
                             preferred_element_type=jnp.float32)
        acc = hj if acc is None else jnp.maximum(acc, hj)
    msg_ref[0] = acc + b2_ref[...]


def _mlp_call(qg, p, w2, b2col, interpret=False):
    return pl.pallas_call(
        _mlp_body,
        grid=(_B, _N // _R),
        in_specs=[
            pl.BlockSpec((1, _K, _R, _CP), lambda b, t: (b, 0, t, 0)),
            pl.BlockSpec((1, _R, _C), lambda b, t: (b, t, 0)),
            pl.BlockSpec((_C, _C), lambda b, t: (0, 0)),
            pl.BlockSpec((_C, 1), lambda b, t: (0, 0)),
        ],
        out_specs=pl.BlockSpec((1, _C, _R), lambda b, t: (b, 0, t)),
        out_shape=jax.ShapeDtypeStruct((_B, _C, _N), jnp.float32),
        interpret=interpret,
    )(qg, p, w2, b2col)


def _bn_body(msg_ref, x_ref, gamma_ref, beta_ref, out_ref):
    s = None
    for b in range(_B):
        sb = jnp.sum(msg_ref[b], axis=1, keepdims=True)
        s = sb if s is None else s + sb
    mean = s * (1.0 / (_B * _N))
    v = None
    for b in range(_B):
        c = msg_ref[b] - mean
        vb = jnp.sum(c * c, axis=1, keepdims=True)
        v = vb if v is None else v + vb
    var = v * (1.0 / (_B * _N))
    scale = lax.rsqrt(var + 1e-5) * gamma_ref[...]
    shift = beta_ref[...] - mean * scale
    for b in range(_B):
        out_ref[b] = jnp.maximum(msg_ref[b] * scale + shift + x_ref[b], 0.0)


def _bn_call(msg, xr, gammacol, betacol, interpret=False):
    return pl.pallas_call(
        _bn_body,
        out_shape=jax.ShapeDtypeStruct((_B, _C, _N), jnp.float32),
        interpret=interpret,
    )(msg, xr, gammacol, betacol)


def kernel(x, W1, b1, W2, b2, gamma, beta):
    B, C, H, W = x.shape
    xr = x.reshape(B, C, H * W)
    w1a = W1[:, :C]
    w1b = W1[:, C:]
    idx, p, q = _prep_call(xr, w1a, w1b, b1.reshape(1, C))
    i = jnp.arange(_TOT, dtype=jnp.int32)
    pos = (i // (_N * _K)) * (_K * _N) + (i % _K) * _N + (i // _K) % _N
    qg = _gather_call()(q.reshape(B * H * W, _CP), idx.reshape(-1), pos)
    msg = _mlp_call(qg.reshape(B, _K, H * W, _CP), p, W2, b2.reshape(C, 1))
    out = _bn_call(msg, xr, gamma.reshape(C, 1), beta.reshape(C, 1))
    return out.reshape(B, C, H, W)


# speedup vs baseline: 1.3417x; 1.0419x over previous
import functools

import jax
import jax.numpy as jnp
from jax import lax
from jax.experimental import pallas as pl
from jax.experimental.pallas import tpu as pltpu
from jax.experimental.pallas import tpu_sc as plsc

_B, _C, _N, _K = 8, 96, 1024, 9
_CP = 128
_KP = 16
_M = 256
_R = 256


def _prep_body(xf_ref, xc_ref, a_ref, bm_ref, b1_ref, idx_ref, p_ref, q_ref):
    b = pl.program_id(0)
    xf = xf_ref[0]
    xc = xc_ref[0]
    sq = jnp.sum(xf * xf, axis=0, keepdims=True)
    g = lax.dot_general(xc, xf, (((0,), (0,)), ((), ())),
                        preferred_element_type=jnp.float32)
    score = sq - 2.0 * g
    iota_f = lax.broadcasted_iota(jnp.int32, (_M, _N), 1).astype(jnp.float32)
    cols = []
    m = jnp.min(score, axis=1, keepdims=True)
    for j in range(_K):
        cmp = score == m
        cols.append(jnp.min(jnp.where(cmp, iota_f, 65536.0), axis=1,
                            keepdims=True).astype(jnp.int32))
        if j + 1 < _K:
            score = jnp.where(cmp, jnp.inf, score)
            m = jnp.min(score, axis=1, keepdims=True)
    idx_ref[0] = jnp.concatenate(cols, axis=1) + b * _N
    wc = a_ref[...] - bm_ref[...]
    p_ref[0] = lax.dot_general(xc, wc, (((0,), (1,)), ((), ())),
                               preferred_element_type=jnp.float32) + b1_ref[...]
    qv = lax.dot_general(xc, bm_ref[...], (((0,), (1,)), ((), ())),
                         preferred_element_type=jnp.float32)
    q_ref[0] = jnp.concatenate(
        [qv, jnp.zeros((_M, _CP - _C), jnp.float32)], axis=1)


def _prep_call(xr, w1a, w1b, b1row, interpret=False):
    return pl.pallas_call(
        _prep_body,
        grid=(_B, _N // _M),
        in_specs=[
            pl.BlockSpec((1, _C, _N), lambda b, t: (b, 0, 0)),
            pl.BlockSpec((1, _C, _M), lambda b, t: (b, 0, t)),
            pl.BlockSpec((_C, _C), lambda b, t: (0, 0)),
            pl.BlockSpec((_C, _C), lambda b, t: (0, 0)),
            pl.BlockSpec((1, _C), lambda b, t: (0, 0)),
        ],
        out_specs=[
            pl.BlockSpec((1, _M, _K), lambda b, t: (b, t, 0)),
            pl.BlockSpec((1, _M, _C), lambda b, t: (b, t, 0)),
            pl.BlockSpec((1, _M, _CP), lambda b, t: (b, t, 0)),
        ],
        out_shape=[
            jax.ShapeDtypeStruct((_B, _N, _K), jnp.int32),
            jax.ShapeDtypeStruct((_B, _N, _C), jnp.float32),
            jax.ShapeDtypeStruct((_B, _N, _CP), jnp.float32),
        ],
        interpret=interpret,
    )(xr, xr, w1a, w1b, b1row)


_TOT = _B * _N * _K
_NW = 32
_PW = _TOT // _NW
_CH = 128
_NCH = _PW // _CH


_NB = 3
_NG = _NCH // _NB


def _gather_body(q_hbm, idx_hbm, pos_hbm, out_hbm, idx_v, pos_v, rows_v,
                 gsem, ssem0, ssem1):
    wid = lax.axis_index("s") * 2 + lax.axis_index("c")
    pltpu.sync_copy(idx_hbm.at[pl.ds(wid * _NCH, _NCH)], idx_v)
    pltpu.sync_copy(pos_hbm.at[pl.ds(wid * _NCH, _NCH)], pos_v)
    ssems = (ssem0, ssem1)
    pending = [[], []]
    for g in range(_NG):
        buf = g % 2
        for d in pending[buf]:
            d.wait()
        pending[buf] = []
        gds = []
        for i in range(_NB):
            c = g * _NB + i
            gds.append(pltpu.async_copy(q_hbm.at[idx_v.at[c, 0]],
                                        rows_v.at[buf, i], gsem))
        for d in gds:
            d.wait()
        for i in range(_NB):
            c = g * _NB + i
            pending[buf].append(
                pltpu.async_copy(rows_v.at[buf, i],
                                 out_hbm.at[pos_v.at[c, 0]], ssems[buf]))
    for par in (0, 1):
        for d in pending[par]:
            d.wait()


@functools.cache
def _gather_call():
    return pl.kernel(
        _gather_body,
        out_type=jax.ShapeDtypeStruct((_TOT, _CP), jnp.float32),
        mesh=plsc.VectorSubcoreMesh(core_axis_name="c", subcore_axis_name="s"),
        scratch_types=[
            pltpu.VMEM((_NCH, 1, _CH), jnp.int32),
            pltpu.VMEM((_NCH, 1, _CH), jnp.int32),
            pltpu.VMEM((2, _NB, _CH, _CP), jnp.float32),
            pltpu.SemaphoreType.DMA,
            pltpu.SemaphoreType.DMA,
            pltpu.SemaphoreType.DMA,
        ],
    )


def _mlp_body(qg_ref, p_ref, w2_ref, b2_ref, msg_ref):
    p = p_ref[0]
    w2 = w2_ref[...]
    acc = None
    for j in range(_K):
        v = p + qg_ref[0, j, :, :_C]
        v = jnp.where(v >= 0, v, 0.2 * v)
        hj = lax.dot_general(w2, v, (((1,), (1,)), ((), ())),
                             preferred_element_type=jnp.float32)
        acc = hj if acc is None else jnp.maximum(acc, hj)
    msg_ref[0] = acc + b2_ref[...]


def _mlp_call(qg, p, w2, b2col, interpret=False):
    return pl.pallas_call(
        _mlp_body,
        grid=(_B, _N // _R),
        in_specs=[
            pl.BlockSpec((1, _K, _R, _CP), lambda b, t: (b, 0, t, 0)),
            pl.BlockSpec((1, _R, _C), lambda b, t: (b, t, 0)),
            pl.BlockSpec((_C, _C), lambda b, t: (0, 0)),
            pl.BlockSpec((_C, 1), lambda b, t: (0, 0)),
        ],
        out_specs=pl.BlockSpec((1, _C, _R), lambda b, t: (b, 0, t)),
        out_shape=jax.ShapeDtypeStruct((_B, _C, _N), jnp.float32),
        interpret=interpret,
    )(qg, p, w2, b2col)


def _bn_body(msg_ref, x_ref, gamma_ref, beta_ref, out_ref):
    s = None
    for b in range(_B):
        sb = jnp.sum(msg_ref[b], axis=1, keepdims=True)
        s = sb if s is None else s + sb
    mean = s * (1.0 / (_B * _N))
    v = None
    for b in range(_B):
        c = msg_ref[b] - mean
        vb = jnp.sum(c * c, axis=1, keepdims=True)
        v = vb if v is None else v + vb
    var = v * (1.0 / (_B * _N))
    scale = lax.rsqrt(var + 1e-5) * gamma_ref[...]
    shift = beta_ref[...] - mean * scale
    for b in range(_B):
        out_ref[b] = jnp.maximum(msg_ref[b] * scale + shift + x_ref[b], 0.0)


def _bn_call(msg, xr, gammacol, betacol, interpret=False):
    return pl.pallas_call(
        _bn_body,
        out_shape=jax.ShapeDtypeStruct((_B, _C, _N), jnp.float32),
        interpret=interpret,
    )(msg, xr, gammacol, betacol)


def kernel(x, W1, b1, W2, b2, gamma, beta):
    B, C, H, W = x.shape
    xr = x.reshape(B, C, H * W)
    w1a = W1[:, :C]
    w1b = W1[:, C:]
    idx, p, q = _prep_call(xr, w1a, w1b, b1.reshape(1, C))
    i = jnp.arange(_TOT, dtype=jnp.int32)
    pos = (i // (_N * _K)) * (_K * _N) + (i % _K) * _N + (i // _K) % _N
    qg = _gather_call()(q.reshape(B * H * W, _CP), idx.reshape(-1, 1, _CH),
                        pos.reshape(-1, 1, _CH))
    msg = _mlp_call(qg.reshape(B, _K, H * W, _CP), p, W2, b2.reshape(C, 1))
    out = _bn_call(msg, xr, gamma.reshape(C, 1), beta.reshape(C, 1))
    return out.reshape(B, C, H, W)


# speedup vs baseline: 1.3587x; 1.0127x over previous
import functools

import jax
import jax.numpy as jnp
from jax import lax
from jax.experimental import pallas as pl
from jax.experimental.pallas import tpu as pltpu
from jax.experimental.pallas import tpu_sc as plsc

_B, _C, _N, _K = 8, 96, 1024, 9
_CP = 128
_KP = 16
_M = 512
_R = 256


def _prep_body(xf_ref, xc_ref, a_ref, bm_ref, b1_ref, idx_ref, p_ref, q_ref):
    b = pl.program_id(0)
    xf = xf_ref[0]
    xc = xc_ref[0]
    sq = jnp.sum(xf * xf, axis=0, keepdims=True)
    g = lax.dot_general(xc, xf, (((0,), (0,)), ((), ())),
                        preferred_element_type=jnp.float32)
    score = sq - 2.0 * g
    iota_f = lax.broadcasted_iota(jnp.int32, (_M, _N), 1).astype(jnp.float32)
    cols = []
    m = jnp.min(score, axis=1, keepdims=True)
    for j in range(_K):
        cmp = score == m
        cols.append(jnp.min(jnp.where(cmp, iota_f, 65536.0), axis=1,
                            keepdims=True).astype(jnp.int32))
        if j + 1 < _K:
            score = jnp.where(cmp, jnp.inf, score)
            m = jnp.min(score, axis=1, keepdims=True)
    idx_ref[0] = jnp.concatenate(cols, axis=1) + b * _N
    wc = a_ref[...] - bm_ref[...]
    p_ref[0] = lax.dot_general(xc, wc, (((0,), (1,)), ((), ())),
                               preferred_element_type=jnp.float32) + b1_ref[...]
    qv = lax.dot_general(xc, bm_ref[...], (((0,), (1,)), ((), ())),
                         preferred_element_type=jnp.float32)
    q_ref[0] = jnp.concatenate(
        [qv, jnp.zeros((_M, _CP - _C), jnp.float32)], axis=1)


def _prep_call(xr, w1a, w1b, b1row, interpret=False):
    return pl.pallas_call(
        _prep_body,
        grid=(_B, _N // _M),
        in_specs=[
            pl.BlockSpec((1, _C, _N), lambda b, t: (b, 0, 0)),
            pl.BlockSpec((1, _C, _M), lambda b, t: (b, 0, t)),
            pl.BlockSpec((_C, _C), lambda b, t: (0, 0)),
            pl.BlockSpec((_C, _C), lambda b, t: (0, 0)),
            pl.BlockSpec((1, _C), lambda b, t: (0, 0)),
        ],
        out_specs=[
            pl.BlockSpec((1, _M, _K), lambda b, t: (b, t, 0)),
            pl.BlockSpec((1, _M, _C), lambda b, t: (b, t, 0)),
            pl.BlockSpec((1, _M, _CP), lambda b, t: (b, t, 0)),
        ],
        out_shape=[
            jax.ShapeDtypeStruct((_B, _N, _K), jnp.int32),
            jax.ShapeDtypeStruct((_B, _N, _C), jnp.float32),
            jax.ShapeDtypeStruct((_B, _N, _CP), jnp.float32),
        ],
        interpret=interpret,
    )(xr, xr, w1a, w1b, b1row)


_TOT = _B * _N * _K
_NW = 32
_PW = _TOT // _NW
_CH = 128
_NCH = _PW // _CH


_NB = 3
_NG = _NCH // _NB


def _gather_body(q_hbm, idx_hbm, pos_hbm, out_hbm, idx_v, pos_v, rows_v,
                 gsem, ssem0, ssem1):
    wid = lax.axis_index("s") * 2 + lax.axis_index("c")
    pltpu.sync_copy(idx_hbm.at[pl.ds(wid * _NCH, _NCH)], idx_v)
    pltpu.sync_copy(pos_hbm.at[pl.ds(wid * _NCH, _NCH)], pos_v)
    ssems = (ssem0, ssem1)
    pending = [[], []]
    for g in range(_NG):
        buf = g % 2
        for d in pending[buf]:
            d.wait()
        pending[buf] = []
        gds = []
        for i in range(_NB):
            c = g * _NB + i
            gds.append(pltpu.async_copy(q_hbm.at[idx_v.at[c, 0]],
                                        rows_v.at[buf, i], gsem))
        for d in gds:
            d.wait()
        for i in range(_NB):
            c = g * _NB + i
            pending[buf].append(
                pltpu.async_copy(rows_v.at[buf, i],
                                 out_hbm.at[pos_v.at[c, 0]], ssems[buf]))
    for par in (0, 1):
        for d in pending[par]:
            d.wait()


@functools.cache
def _gather_call():
    return pl.kernel(
        _gather_body,
        out_type=jax.ShapeDtypeStruct((_TOT, _CP), jnp.float32),
        mesh=plsc.VectorSubcoreMesh(core_axis_name="c", subcore_axis_name="s"),
        scratch_types=[
            pltpu.VMEM((_NCH, 1, _CH), jnp.int32),
            pltpu.VMEM((_NCH, 1, _CH), jnp.int32),
            pltpu.VMEM((2, _NB, _CH, _CP), jnp.float32),
            pltpu.SemaphoreType.DMA,
            pltpu.SemaphoreType.DMA,
            pltpu.SemaphoreType.DMA,
        ],
    )


def _mlp_body(qg_ref, p_ref, w2_ref, b2_ref, msg_ref):
    p = p_ref[0]
    w2 = w2_ref[...]
    acc = None
    for j in range(_K):
        v = p + qg_ref[0, j, :, :_C]
        v = jnp.where(v >= 0, v, 0.2 * v)
        hj = lax.dot_general(w2, v, (((1,), (1,)), ((), ())),
                             preferred_element_type=jnp.float32)
        acc = hj if acc is None else jnp.maximum(acc, hj)
    msg_ref[0] = acc + b2_ref[...]


def _mlp_call(qg, p, w2, b2col, interpret=False):
    return pl.pallas_call(
        _mlp_body,
        grid=(_B, _N // _R),
        in_specs=[
            pl.BlockSpec((1, _K, _R, _CP), lambda b, t: (b, 0, t, 0)),
            pl.BlockSpec((1, _R, _C), lambda b, t: (b, t, 0)),
            pl.BlockSpec((_C, _C), lambda b, t: (0, 0)),
            pl.BlockSpec((_C, 1), lambda b, t: (0, 0)),
        ],
        out_specs=pl.BlockSpec((1, _C, _R), lambda b, t: (b, 0, t)),
        out_shape=jax.ShapeDtypeStruct((_B, _C, _N), jnp.float32),
        interpret=interpret,
    )(qg, p, w2, b2col)


def _bn_body(msg_ref, x_ref, gamma_ref, beta_ref, out_ref):
    s = None
    for b in range(_B):
        sb = jnp.sum(msg_ref[b], axis=1, keepdims=True)
        s = sb if s is None else s + sb
    mean = s * (1.0 / (_B * _N))
    v = None
    for b in range(_B):
        c = msg_ref[b] - mean
        vb = jnp.sum(c * c, axis=1, keepdims=True)
        v = vb if v is None else v + vb
    var = v * (1.0 / (_B * _N))
    scale = lax.rsqrt(var + 1e-5) * gamma_ref[...]
    shift = beta_ref[...] - mean * scale
    for b in range(_B):
        out_ref[b] = jnp.maximum(msg_ref[b] * scale + shift + x_ref[b], 0.0)


def _bn_call(msg, xr, gammacol, betacol, interpret=False):
    return pl.pallas_call(
        _bn_body,
        out_shape=jax.ShapeDtypeStruct((_B, _C, _N), jnp.float32),
        interpret=interpret,
    )(msg, xr, gammacol, betacol)


def kernel(x, W1, b1, W2, b2, gamma, beta):
    B, C, H, W = x.shape
    xr = x.reshape(B, C, H * W)
    w1a = W1[:, :C]
    w1b = W1[:, C:]
    idx, p, q = _prep_call(xr, w1a, w1b, b1.reshape(1, C))
    i = jnp.arange(_TOT, dtype=jnp.int32)
    pos = (i // (_N * _K)) * (_K * _N) + (i % _K) * _N + (i // _K) % _N
    qg = _gather_call()(q.reshape(B * H * W, _CP), idx.reshape(-1, 1, _CH),
                        pos.reshape(-1, 1, _CH))
    msg = _mlp_call(qg.reshape(B, _K, H * W, _CP), p, W2, b2.reshape(C, 1))
    out = _bn_call(msg, xr, gamma.reshape(C, 1), beta.reshape(C, 1))
    return out.reshape(B, C, H, W)


# speedup vs baseline: 1.4252x; 1.0489x over previous
import functools

import jax
import jax.numpy as jnp
from jax import lax
from jax.experimental import pallas as pl
from jax.experimental.pallas import tpu as pltpu
from jax.experimental.pallas import tpu_sc as plsc

_B, _C, _N, _K = 8, 96, 1024, 9
_CP = 128
_KP = 16
_M = 512
_R = 512


def _prep_body(xf_ref, xc_ref, a_ref, bm_ref, b1_ref, idx_ref, p_ref, q_ref):
    b = pl.program_id(0)
    xf = xf_ref[0]
    xc = xc_ref[0]
    sq = jnp.sum(xf * xf, axis=0, keepdims=True)
    g = lax.dot_general(xc, xf, (((0,), (0,)), ((), ())),
                        preferred_element_type=jnp.float32)
    score = sq - 2.0 * g
    iota_f = lax.broadcasted_iota(jnp.int32, (_M, _N), 1).astype(jnp.float32)
    cols = []
    m = jnp.min(score, axis=1, keepdims=True)
    for j in range(_K):
        cmp = score == m
        cols.append(jnp.min(jnp.where(cmp, iota_f, 65536.0), axis=1,
                            keepdims=True).astype(jnp.int32))
        if j + 1 < _K:
            score = jnp.where(cmp, jnp.inf, score)
            m = jnp.min(score, axis=1, keepdims=True)
    idx_ref[0] = jnp.concatenate(cols, axis=1) + b * _N
    wc = a_ref[...] - bm_ref[...]
    pad = jnp.zeros((_M, _CP - _C), jnp.float32)
    pv = lax.dot_general(xc, wc, (((0,), (1,)), ((), ())),
                         preferred_element_type=jnp.float32) + b1_ref[...]
    p_ref[0] = jnp.concatenate([pv, pad], axis=1)
    qv = lax.dot_general(xc, bm_ref[...], (((0,), (1,)), ((), ())),
                         preferred_element_type=jnp.float32)
    q_ref[0] = jnp.concatenate([qv, pad], axis=1)


def _prep_call(xr, w1a, w1b, b1row, interpret=False):
    return pl.pallas_call(
        _prep_body,
        grid=(_B, _N // _M),
        in_specs=[
            pl.BlockSpec((1, _C, _N), lambda b, t: (b, 0, 0)),
            pl.BlockSpec((1, _C, _M), lambda b, t: (b, 0, t)),
            pl.BlockSpec((_C, _C), lambda b, t: (0, 0)),
            pl.BlockSpec((_C, _C), lambda b, t: (0, 0)),
            pl.BlockSpec((1, _C), lambda b, t: (0, 0)),
        ],
        out_specs=[
            pl.BlockSpec((1, _M, _K), lambda b, t: (b, t, 0)),
            pl.BlockSpec((1, _M, _CP), lambda b, t: (b, t, 0)),
            pl.BlockSpec((1, _M, _CP), lambda b, t: (b, t, 0)),
        ],
        out_shape=[
            jax.ShapeDtypeStruct((_B, _N, _K), jnp.int32),
            jax.ShapeDtypeStruct((_B, _N, _CP), jnp.float32),
            jax.ShapeDtypeStruct((_B, _N, _CP), jnp.float32),
        ],
        interpret=interpret,
    )(xr, xr, w1a, w1b, b1row)


_TOT = _B * _N * _K
_NW = 32
_PW = _TOT // _NW
_CH = 128
_NCH = _PW // _CH


_NB = 3
_NG = _NCH // _NB


def _gather_body(q_hbm, idx_hbm, pos_hbm, out_hbm, idx_v, pos_v, rows_v,
                 gsem, ssem0, ssem1):
    wid = lax.axis_index("s") * 2 + lax.axis_index("c")
    pltpu.sync_copy(idx_hbm.at[pl.ds(wid * _NCH, _NCH)], idx_v)
    pltpu.sync_copy(pos_hbm.at[pl.ds(wid * _NCH, _NCH)], pos_v)
    ssems = (ssem0, ssem1)
    pending = [[], []]
    for g in range(_NG):
        buf = g % 2
        for d in pending[buf]:
            d.wait()
        pending[buf] = []
        gds = []
        for i in range(_NB):
            c = g * _NB + i
            gds.append(pltpu.async_copy(q_hbm.at[idx_v.at[c, 0]],
                                        rows_v.at[buf, i], gsem))
        for d in gds:
            d.wait()
        for i in range(_NB):
            c = g * _NB + i
            pending[buf].append(
                pltpu.async_copy(rows_v.at[buf, i],
                                 out_hbm.at[pos_v.at[c, 0]], ssems[buf]))
    for par in (0, 1):
        for d in pending[par]:
            d.wait()


@functools.cache
def _gather_call():
    return pl.kernel(
        _gather_body,
        out_type=jax.ShapeDtypeStruct((_TOT, _CP), jnp.float32),
        mesh=plsc.VectorSubcoreMesh(core_axis_name="c", subcore_axis_name="s"),
        scratch_types=[
            pltpu.VMEM((_NCH, 1, _CH), jnp.int32),
            pltpu.VMEM((_NCH, 1, _CH), jnp.int32),
            pltpu.VMEM((2, _NB, _CH, _CP), jnp.float32),
            pltpu.SemaphoreType.DMA,
            pltpu.SemaphoreType.DMA,
            pltpu.SemaphoreType.DMA,
        ],
    )


def _mlp_body(qg_ref, p_ref, w2_ref, b2_ref, msg_ref):
    p = p_ref[0]
    w2p = jnp.concatenate(
        [w2_ref[...], jnp.zeros((_C, _CP - _C), jnp.float32)], axis=1)
    acc = None
    for j in range(_K):
        v = p + qg_ref[0, j]
        v = jnp.where(v >= 0, v, 0.2 * v)
        hj = lax.dot_general(w2p, v, (((1,), (1,)), ((), ())),
                             preferred_element_type=jnp.float32)
        acc = hj if acc is None else jnp.maximum(acc, hj)
    msg_ref[0] = acc + b2_ref[...]


def _mlp_call(qg, p, w2, b2col, interpret=False):
    return pl.pallas_call(
        _mlp_body,
        grid=(_B, _N // _R),
        in_specs=[
            pl.BlockSpec((1, _K, _R, _CP), lambda b, t: (b, 0, t, 0)),
            pl.BlockSpec((1, _R, _CP), lambda b, t: (b, t, 0)),
            pl.BlockSpec((_C, _C), lambda b, t: (0, 0)),
            pl.BlockSpec((_C, 1), lambda b, t: (0, 0)),
        ],
        out_specs=pl.BlockSpec((1, _C, _R), lambda b, t: (b, 0, t)),
        out_shape=jax.ShapeDtypeStruct((_B, _C, _N), jnp.float32),
        interpret=interpret,
    )(qg, p, w2, b2col)


def _bn_body(msg_ref, x_ref, gamma_ref, beta_ref, out_ref):
    s = None
    for b in range(_B):
        sb = jnp.sum(msg_ref[b], axis=1, keepdims=True)
        s = sb if s is None else s + sb
    mean = s * (1.0 / (_B * _N))
    v = None
    for b in range(_B):
        c = msg_ref[b] - mean
        vb = jnp.sum(c * c, axis=1, keepdims=True)
        v = vb if v is None else v + vb
    var = v * (1.0 / (_B * _N))
    scale = lax.rsqrt(var + 1e-5) * gamma_ref[...]
    shift = beta_ref[...] - mean * scale
    for b in range(_B):
        out_ref[b] = jnp.maximum(msg_ref[b] * scale + shift + x_ref[b], 0.0)


def _bn_call(msg, xr, gammacol, betacol, interpret=False):
    return pl.pallas_call(
        _bn_body,
        out_shape=jax.ShapeDtypeStruct((_B, _C, _N), jnp.float32),
        interpret=interpret,
    )(msg, xr, gammacol, betacol)


def kernel(x, W1, b1, W2, b2, gamma, beta):
    B, C, H, W = x.shape
    xr = x.reshape(B, C, H * W)
    w1a = W1[:, :C]
    w1b = W1[:, C:]
    idx, p, q = _prep_call(xr, w1a, w1b, b1.reshape(1, C))
    i = jnp.arange(_TOT, dtype=jnp.int32)
    pos = (i // (_N * _K)) * (_K * _N) + (i % _K) * _N + (i // _K) % _N
    qg = _gather_call()(q.reshape(B * H * W, _CP), idx.reshape(-1, 1, _CH),
                        pos.reshape(-1, 1, _CH))
    msg = _mlp_call(qg.reshape(B, _K, H * W, _CP), p, W2, b2.reshape(C, 1))
    out = _bn_call(msg, xr, gamma.reshape(C, 1), beta.reshape(C, 1))
    return out.reshape(B, C, H, W)


# speedup vs baseline: 1.5366x; 1.0782x over previous
import functools

import jax
import jax.numpy as jnp
from jax import lax
from jax.experimental import pallas as pl
from jax.experimental.pallas import tpu as pltpu
from jax.experimental.pallas import tpu_sc as plsc

_B, _C, _N, _K = 8, 96, 1024, 9
_CP = 128
_M = 512
_R = 512
_BH = _B // 2


def _prep_body(xf_ref, xc_ref, a_ref, bm_ref, b1_ref, idx_ref, p_ref, q_ref):
    b = pl.program_id(0)
    xf = xf_ref[0]
    xc = xc_ref[0]
    sq = jnp.sum(xf * xf, axis=0, keepdims=True)
    g = lax.dot_general(xc, xf, (((0,), (0,)), ((), ())),
                        preferred_element_type=jnp.float32)
    score = sq - 2.0 * g
    iota_f = lax.broadcasted_iota(jnp.int32, (_M, _N), 1).astype(jnp.float32)
    cols = []
    m = jnp.min(score, axis=1, keepdims=True)
    for j in range(_K):
        cmp = score == m
        cols.append(jnp.min(jnp.where(cmp, iota_f, 65536.0), axis=1,
                            keepdims=True).astype(jnp.int32))
        if j + 1 < _K:
            score = jnp.where(cmp, jnp.inf, score)
            m = jnp.min(score, axis=1, keepdims=True)
    idx_ref[0] = jnp.concatenate(cols, axis=1) + b * _N
    wc = a_ref[...] - bm_ref[...]
    pad = jnp.zeros((_M, _CP - _C), jnp.float32)
    pv = lax.dot_general(xc, wc, (((0,), (1,)), ((), ())),
                         preferred_element_type=jnp.float32) + b1_ref[...]
    p_ref[0] = jnp.concatenate([pv, pad], axis=1)
    qv = lax.dot_general(xc, bm_ref[...], (((0,), (1,)), ((), ())),
                         preferred_element_type=jnp.float32)
    q_ref[0] = jnp.concatenate([qv, pad], axis=1)


def _prep_call(xr, w1a, w1b, b1row, off, interpret=False):
    return pl.pallas_call(
        _prep_body,
        grid=(_BH, _N // _M),
        in_specs=[
            pl.BlockSpec((1, _C, _N), lambda b, t: (b + off, 0, 0)),
            pl.BlockSpec((1, _C, _M), lambda b, t: (b + off, 0, t)),
            pl.BlockSpec((_C, _C), lambda b, t: (0, 0)),
            pl.BlockSpec((_C, _C), lambda b, t: (0, 0)),
            pl.BlockSpec((1, _C), lambda b, t: (0, 0)),
        ],
        out_specs=[
            pl.BlockSpec((1, _M, _K), lambda b, t: (b, t, 0)),
            pl.BlockSpec((1, _M, _CP), lambda b, t: (b, t, 0)),
            pl.BlockSpec((1, _M, _CP), lambda b, t: (b, t, 0)),
        ],
        out_shape=[
            jax.ShapeDtypeStruct((_BH, _N, _K), jnp.int32),
            jax.ShapeDtypeStruct((_BH, _N, _CP), jnp.float32),
            jax.ShapeDtypeStruct((_BH, _N, _CP), jnp.float32),
        ],
        interpret=interpret,
    )(xr, xr, w1a, w1b, b1row)


_TOTH = _BH * _N * _K
_NW = 32
_PWH = _TOTH // _NW
_CH = 128
_NCHH = _PWH // _CH
_NB = 3
_NGH = _NCHH // _NB


def _gather_body(q_hbm, idx_hbm, pos_hbm, out_hbm, idx_v, pos_v, rows_v,
                 gsem, ssem0, ssem1):
    wid = lax.axis_index("s") * 2 + lax.axis_index("c")
    pltpu.sync_copy(idx_hbm.at[pl.ds(wid * _NCHH, _NCHH)], idx_v)
    pltpu.sync_copy(pos_hbm.at[pl.ds(wid * _NCHH, _NCHH)], pos_v)
    ssems = (ssem0, ssem1)
    pending = [[], []]
    for g in range(_NGH):
        buf = g % 2
        for d in pending[buf]:
            d.wait()
        pending[buf] = []
        gds = []
        for i in range(_NB):
            c = g * _NB + i
            gds.append(pltpu.async_copy(q_hbm.at[idx_v.at[c, 0]],
                                        rows_v.at[buf, i], gsem))
        for d in gds:
            d.wait()
        for i in range(_NB):
            c = g * _NB + i
            pending[buf].append(
                pltpu.async_copy(rows_v.at[buf, i],
                                 out_hbm.at[pos_v.at[c, 0]], ssems[buf]))
    for par in (0, 1):
        for d in pending[par]:
            d.wait()


@functools.cache
def _gather_call():
    return pl.kernel(
        _gather_body,
        out_type=jax.ShapeDtypeStruct((_TOTH, _CP), jnp.float32),
        mesh=plsc.VectorSubcoreMesh(core_axis_name="c", subcore_axis_name="s"),
        scratch_types=[
            pltpu.VMEM((_NCHH, 1, _CH), jnp.int32),
            pltpu.VMEM((_NCHH, 1, _CH), jnp.int32),
            pltpu.VMEM((2, _NB, _CH, _CP), jnp.float32),
            pltpu.SemaphoreType.DMA,
            pltpu.SemaphoreType.DMA,
            pltpu.SemaphoreType.DMA,
        ],
    )


def _mlp_body(qg_ref, p_ref, w2_ref, b2_ref, msg_ref):
    p = p_ref[0]
    w2p = jnp.concatenate(
        [w2_ref[...], jnp.zeros((_C, _CP - _C), jnp.float32)], axis=1)
    acc = None
    for j in range(_K):
        v = p + qg_ref[0, j]
        v = jnp.where(v >= 0, v, 0.2 * v)
        hj = lax.dot_general(w2p, v, (((1,), (1,)), ((), ())),
                             preferred_element_type=jnp.float32)
        acc = hj if acc is None else jnp.maximum(acc, hj)
    msg_ref[0] = acc + b2_ref[...]


def _mlp_call(qg, p, w2, b2col, interpret=False):
    return pl.pallas_call(
        _mlp_body,
        grid=(_BH, _N // _R),
        in_specs=[
            pl.BlockSpec((1, _K, _R, _CP), lambda b, t: (b, 0, t, 0)),
            pl.BlockSpec((1, _R, _CP), lambda b, t: (b, t, 0)),
            pl.BlockSpec((_C, _C), lambda b, t: (0, 0)),
            pl.BlockSpec((_C, 1), lambda b, t: (0, 0)),
        ],
        out_specs=pl.BlockSpec((1, _C, _R), lambda b, t: (b, 0, t)),
        out_shape=jax.ShapeDtypeStruct((_BH, _C, _N), jnp.float32),
        interpret=interpret,
    )(qg, p, w2, b2col)


def _bn_body(msg0_ref, msg1_ref, x_ref, gamma_ref, beta_ref, out_ref):
    halves = (msg0_ref, msg1_ref)
    s = None
    for h in range(2):
        for b in range(_BH):
            sb = jnp.sum(halves[h][b], axis=1, keepdims=True)
            s = sb if s is None else s + sb
    mean = s * (1.0 / (_B * _N))
    v = None
    for h in range(2):
        for b in range(_BH):
            c = halves[h][b] - mean
            vb = jnp.sum(c * c, axis=1, keepdims=True)
            v = vb if v is None else v + vb
    var = v * (1.0 / (_B * _N))
    scale = lax.rsqrt(var + 1e-5) * gamma_ref[...]
    shift = beta_ref[...] - mean * scale
    for h in range(2):
        for b in range(_BH):
            out_ref[h * _BH + b] = jnp.maximum(
                halves[h][b] * scale + shift + x_ref[h * _BH + b], 0.0)


def _bn_call(msg0, msg1, xr, gammacol, betacol, interpret=False):
    return pl.pallas_call(
        _bn_body,
        out_shape=jax.ShapeDtypeStruct((_B, _C, _N), jnp.float32),
        interpret=interpret,
    )(msg0, msg1, xr, gammacol, betacol)


def _half(xr, w1a, w1b, b1row, W2, b2col, pos, off):
    idx, p, q = _prep_call(xr, w1a, w1b, b1row, off)
    qg = _gather_call()(q.reshape(_BH * _N, _CP), idx.reshape(-1, 1, _CH),
                        pos)
    return _mlp_call(qg.reshape(_BH, _K, _N, _CP), p, W2, b2col)


def kernel(x, W1, b1, W2, b2, gamma, beta):
    B, C, H, W = x.shape
    xr = x.reshape(B, C, H * W)
    w1a = W1[:, :C]
    w1b = W1[:, C:]
    b1row = b1.reshape(1, C)
    b2col = b2.reshape(C, 1)
    i = jnp.arange(_TOTH, dtype=jnp.int32)
    pos = ((i // (_N * _K)) * (_K * _N) + (i % _K) * _N + (i // _K) % _N)
    pos = pos.reshape(-1, 1, _CH)
    msg0 = _half(xr, w1a, w1b, b1row, W2, b2col, pos, 0)
    msg1 = _half(xr, w1a, w1b, b1row, W2, b2col, pos, _BH)
    out = _bn_call(msg0, msg1, xr, gamma.reshape(C, 1), beta.reshape(C, 1))
    return out.reshape(B, C, H, W)
